# baseline (speedup 1.0000x reference)
import functools

import jax
import jax.numpy as jnp
from jax import lax
from jax.experimental import pallas as pl
from jax.experimental.pallas import tpu as pltpu
from jax.experimental.pallas import tpu_sc as plsc

N_NODES = 10000
D_IN = 128
HID = 256
D_OUT = 128
N_SUBCORES = 16
CHUNK = 96
NBUF = 3
EBUF = 4
GROUP = 12


_BLK_M = 2000


def _mm1_body(x_ref, w_ref, o_ref):
    o_ref[0] = jnp.dot(x_ref[...], w_ref[...], preferred_element_type=jnp.float32)


def _matmul_split(x, w):
    n, k = x.shape
    fh = w.shape[1] // 2
    grid = (n // _BLK_M, 2)
    return pl.pallas_call(
        _mm1_body,
        grid=grid,
        in_specs=[
            pl.BlockSpec((_BLK_M, k), lambda i, c: (i, 0)),
            pl.BlockSpec((k, fh), lambda i, c: (0, c)),
        ],
        out_specs=pl.BlockSpec((1, _BLK_M, fh), lambda i, c: (c, i, 0)),
        out_shape=jax.ShapeDtypeStruct((2, n, fh), jnp.float32),
    )(x, w)


def _grelu(x, ga, gb, gc, gd):
    out = jnp.where(x < 0, ga * x, x)
    out = jnp.where((x >= 0) & (x < gc), gb * x, out)
    out = jnp.where(x >= gc, gd * x, out)
    return out


def _mm2_body(p_ref, agg_ref, w_ref, o_ref):
    ga, gb, gc, gd = p_ref[0], p_ref[1], p_ref[2], p_ref[3]
    a0 = _grelu(agg_ref[0], ga, gb, gc, gd)
    a1 = _grelu(agg_ref[1], ga, gb, gc, gd)
    k = a0.shape[1]
    o_ref[...] = jnp.dot(a0, w_ref[:k, :], preferred_element_type=jnp.float32) + jnp.dot(
        a1, w_ref[k:, :], preferred_element_type=jnp.float32
    )


def _matmul2_full(params, agg, w):
    _, n, kh = agg.shape
    f = w.shape[1]
    grid = (n // _BLK_M,)
    return pl.pallas_call(
        _mm2_body,
        grid=grid,
        in_specs=[
            pl.BlockSpec(memory_space=pltpu.SMEM),
            pl.BlockSpec((2, _BLK_M, kh), lambda i: (0, i, 0)),
            pl.BlockSpec((2 * kh, f), lambda i: (0, 0)),
        ],
        out_specs=pl.BlockSpec((_BLK_M, f), lambda i: (i, 0)),
        out_shape=jax.ShapeDtypeStruct((n, f), jnp.float32),
    )(params, agg, w)


def _sum2_body(in_ref, o_ref):
    o_ref[...] = in_ref[0] + in_ref[1]


def _sum_partials(p):
    _, n, f = p.shape
    grid = (n // _BLK_M,)
    return pl.pallas_call(
        _sum2_body,
        grid=grid,
        in_specs=[pl.BlockSpec((2, _BLK_M, f), lambda i: (0, i, 0))],
        out_specs=pl.BlockSpec((_BLK_M, f), lambda i: (i, 0)),
        out_shape=jax.ShapeDtypeStruct((n, f), jnp.float32),
    )(p)


def _make_sc_agg(f, n_h, e_pad, edge_split):
    n = N_NODES
    n_chunks = e_pad // CHUNK
    cps = n_chunks // (2 * N_SUBCORES if edge_split else N_SUBCORES)
    assert cps % GROUP == 0 and cps >= 2 * GROUP
    n_fill = 10
    rows_per_fill = n // n_fill
    btile = 20

    mesh = plsc.VectorSubcoreMesh(core_axis_name="c", subcore_axis_name="s")

    @functools.partial(
        pl.kernel,
        out_type=jax.ShapeDtypeStruct((2, n, f), jnp.float32),
        mesh=mesh,
        compiler_params=pltpu.CompilerParams(needs_layout_passes=False),
        scratch_types=[
            [pltpu.VMEM((3, CHUNK), jnp.int32) for _ in range(EBUF)],
            [pltpu.VMEM((CHUNK, f), jnp.float32) for _ in range(NBUF)],
            pltpu.VMEM((btile, f), jnp.float32),
            pltpu.VMEM_SHARED((n, f), jnp.float32),
            [pltpu.SemaphoreType.DMA for _ in range(EBUF)],
            [pltpu.SemaphoreType.DMA for _ in range(NBUF)],
            [pltpu.SemaphoreType.DMA for _ in range(NBUF)],
        ],
    )
    def sc_agg(h_hbm, edata_hbm, bias_hbm, out_hbm,
               ebufs, rows, btile_v, acc_sh, esem, gsem, ssem):
        c = lax.axis_index("c")
        s = lax.axis_index("s")

        if edge_split:
            chunk0 = (c * N_SUBCORES + s) * cps
        else:
            chunk0 = s * cps

        @pl.when(s < n_fill)
        def _init():
            pltpu.sync_copy(bias_hbm.at[c], btile_v.at[pl.ds(0, 1)])
            for j in range(f // 16):
                sl = pl.ds(j * 16, 16)
                bv = btile_v[0, sl]
                for r in range(1, btile):
                    btile_v[r, sl] = bv
            for t in range(rows_per_fill // btile):
                pltpu.sync_copy(
                    btile_v, acc_sh.at[pl.ds(s * rows_per_fill + t * btile, btile)]
                )

        plsc.subcore_barrier()

        def start_edges(k, eb):
            pltpu.async_copy(edata_hbm.at[c, chunk0 + k], ebufs[eb], esem[eb])

        def wait_edges(k, eb):
            pltpu.make_async_copy(
                edata_hbm.at[c, chunk0 + k], ebufs[eb], esem[eb]
            ).wait()

        def start_gather(eb, b):
            pltpu.async_copy(h_hbm.at[ebufs[eb].at[0]], rows[b], gsem[b])

        def wait_gather(eb, b):
            pltpu.make_async_copy(h_hbm.at[ebufs[eb].at[0]], rows[b], gsem[b]).wait()

        def start_scatter(eb, b):
            pltpu.async_copy(rows[b], acc_sh.at[ebufs[eb].at[1]], ssem[b], add=True)

        def wait_scatter(eb, b):
            pltpu.make_async_copy(rows[b], acc_sh.at[ebufs[eb].at[1]], ssem[b]).wait()

        def scale(eb, b):
            ebuf_s = ebufs[eb]
            rows_b = rows[b]

            two = jnp.full((16,), 2, jnp.int32)

            def scale_group(g2, inner):
                base_e = g2 * 16
                for e in range(16):
                    row = base_e + e
                    wi = plsc.load_gather(ebuf_s, [two, jnp.full((16,), row, jnp.int32)])
                    ws = plsc.bitcast(wi, jnp.float32)
                    for j in range(f // 16):
                        sl = pl.ds(j * 16, 16)
                        rows_b[row, sl] = rows_b[row, sl] * ws
                return inner

            lax.fori_loop(0, CHUNK // 16, scale_group, 0)

        start_edges(0, 0)
        start_edges(1, 1)
        wait_edges(0, 0)
        start_gather(0, 0)

        def group_body(g, carry):
            for b in range(GROUP):
                k = g * GROUP + b
                b3 = b % NBUF
                b4 = b % EBUF

                @pl.when(k >= 2)
                def _wait_sc():
                    wait_scatter((b + 2) % EBUF, (b + 1) % NBUF)

                @pl.when(k + 2 < cps)
                def _pref():
                    start_edges(k + 2, (b + 2) % EBUF)

                @pl.when(k + 1 < cps)
                def _next_gather():
                    wait_edges(k + 1, (b + 1) % EBUF)
                    start_gather((b + 1) % EBUF, (b + 1) % NBUF)

                wait_gather(b4, b3)
                scale(b4, b3)
                start_scatter(b4, b3)
            return carry

        lax.fori_loop(0, cps // GROUP, group_body, 0)

        wait_scatter((cps - 2) % EBUF, (cps - 2) % NBUF)
        wait_scatter((cps - 1) % EBUF, (cps - 1) % NBUF)

        plsc.subcore_barrier()

        @pl.when(s < n_fill)
        def _drain():
            r0 = s * rows_per_fill
            pltpu.sync_copy(
                acc_sh.at[pl.ds(r0, rows_per_fill)],
                out_hbm.at[c].at[pl.ds(r0, rows_per_fill)],
            )

    return sc_agg


def _pack_edata(src0, src1, dst, w, n_chunks):
    w_bits = lax.bitcast_convert_type(w, jnp.int32)

    def per_core(s):
        return jnp.stack(
            [
                s.reshape(n_chunks, CHUNK),
                dst.reshape(n_chunks, CHUNK),
                w_bits.reshape(n_chunks, CHUNK),
            ],
            axis=1,
        )

    return jnp.stack([per_core(src0), per_core(src1)])


def kernel(x, edge_index, edge_weight, W1, b1, W2, b2, a, b, c, d):
    n = x.shape[0]
    e = edge_index.shape[1]
    quant = 2 * N_SUBCORES * CHUNK * GROUP
    e_pad = ((e + quant - 1) // quant) * quant
    pad = e_pad - e
    n_chunks = e_pad // CHUNK

    src = jnp.concatenate([edge_index[0], jnp.zeros((pad,), jnp.int32)])
    dst = jnp.concatenate([edge_index[1], jnp.zeros((pad,), jnp.int32)])
    w = jnp.concatenate([edge_weight, jnp.zeros((pad,), jnp.float32)])

    edata1 = _pack_edata(src, src + n, dst, w, n_chunks)
    edata2 = _pack_edata(src, src, dst, w, n_chunks)

    h1 = _matmul_split(x, W1)
    agg1 = _make_sc_agg(HID // 2, 2 * n, e_pad, edge_split=False)(
        h1.reshape(2 * n, HID // 2), edata1, b1.reshape(2, 1, HID // 2)
    )

    params = jnp.stack([a, b, c, d])
    h2 = _matmul2_full(params, agg1, W2)
    bias2 = jnp.stack([b2, jnp.zeros_like(b2)]).reshape(2, 1, D_OUT)
    parts = _make_sc_agg(D_OUT, n, e_pad, edge_split=True)(
        h2, edata2, bias2
    )
    return _sum_partials(parts)


# speedup vs baseline: 3.1884x; 3.1884x over previous
import functools

import jax
import jax.numpy as jnp
from jax import lax
from jax.experimental import pallas as pl
from jax.experimental.pallas import tpu as pltpu
from jax.experimental.pallas import tpu_sc as plsc

N_NODES = 10000
D_IN = 128
HID = 256
D_OUT = 128
N_SUBCORES = 16
CHUNK = 96
NBUF = 3
EBUF = 4
GROUP = 12


_BLK_M = 2000


def _mm1_body(x_ref, w_ref, o_ref):
    o_ref[0] = jnp.dot(x_ref[...], w_ref[...], preferred_element_type=jnp.float32)


def _matmul_split(x, w):
    n, k = x.shape
    fh = w.shape[1] // 2
    grid = (n // _BLK_M, 2)
    return pl.pallas_call(
        _mm1_body,
        grid=grid,
        in_specs=[
            pl.BlockSpec((_BLK_M, k), lambda i, c: (i, 0)),
            pl.BlockSpec((k, fh), lambda i, c: (0, c)),
        ],
        out_specs=pl.BlockSpec((1, _BLK_M, fh), lambda i, c: (c, i, 0)),
        out_shape=jax.ShapeDtypeStruct((2, n, fh), jnp.float32),
    )(x, w)


def _grelu(x, ga, gb, gc, gd):
    out = jnp.where(x < 0, ga * x, x)
    out = jnp.where((x >= 0) & (x < gc), gb * x, out)
    out = jnp.where(x >= gc, gd * x, out)
    return out


def _mm2_body(p_ref, agg_ref, w_ref, o_ref):
    ga, gb, gc, gd = p_ref[0], p_ref[1], p_ref[2], p_ref[3]
    a0 = _grelu(agg_ref[0], ga, gb, gc, gd)
    a1 = _grelu(agg_ref[1], ga, gb, gc, gd)
    k = a0.shape[1]
    o_ref[...] = jnp.dot(a0, w_ref[:k, :], preferred_element_type=jnp.float32) + jnp.dot(
        a1, w_ref[k:, :], preferred_element_type=jnp.float32
    )


def _matmul2_full(params, agg, w):
    _, n, kh = agg.shape
    f = w.shape[1]
    grid = (n // _BLK_M,)
    return pl.pallas_call(
        _mm2_body,
        grid=grid,
        in_specs=[
            pl.BlockSpec(memory_space=pltpu.SMEM),
            pl.BlockSpec((2, _BLK_M, kh), lambda i: (0, i, 0)),
            pl.BlockSpec((2 * kh, f), lambda i: (0, 0)),
        ],
        out_specs=pl.BlockSpec((_BLK_M, f), lambda i: (i, 0)),
        out_shape=jax.ShapeDtypeStruct((n, f), jnp.float32),
    )(params, agg, w)


def _sum2_body(in_ref, o_ref):
    o_ref[...] = in_ref[0] + in_ref[1]


def _sum_partials(p):
    _, n, f = p.shape
    grid = (n // _BLK_M,)
    return pl.pallas_call(
        _sum2_body,
        grid=grid,
        in_specs=[pl.BlockSpec((2, _BLK_M, f), lambda i: (0, i, 0))],
        out_specs=pl.BlockSpec((_BLK_M, f), lambda i: (i, 0)),
        out_shape=jax.ShapeDtypeStruct((n, f), jnp.float32),
    )(p)


def _make_sc_agg(f, n_h, e_pad, edge_split):
    n = N_NODES
    n_chunks = e_pad // CHUNK
    cps = n_chunks // (2 * N_SUBCORES if edge_split else N_SUBCORES)
    assert cps % GROUP == 0 and cps >= 2 * GROUP
    n_fill = 10
    rows_per_fill = n // n_fill
    btile = 20

    mesh = plsc.VectorSubcoreMesh(core_axis_name="c", subcore_axis_name="s")

    @functools.partial(
        pl.kernel,
        out_type=jax.ShapeDtypeStruct((2, n, f), jnp.float32),
        mesh=mesh,
        compiler_params=pltpu.CompilerParams(needs_layout_passes=False),
        scratch_types=[
            [pltpu.VMEM((3, CHUNK), jnp.int32) for _ in range(EBUF)],
            [pltpu.VMEM((CHUNK, f), jnp.float32) for _ in range(NBUF)],
            pltpu.VMEM((btile, f), jnp.float32),
            pltpu.VMEM_SHARED((n, f), jnp.float32),
            [pltpu.SemaphoreType.DMA for _ in range(EBUF)],
            [pltpu.SemaphoreType.DMA for _ in range(NBUF)],
            [pltpu.SemaphoreType.DMA for _ in range(NBUF)],
        ],
    )
    def sc_agg(h_hbm, edata_hbm, bias_hbm, out_hbm,
               ebufs, rows, btile_v, acc_sh, esem, gsem, ssem):
        c = lax.axis_index("c")
        s = lax.axis_index("s")

        if edge_split:
            chunk0 = (c * N_SUBCORES + s) * cps
        else:
            chunk0 = s * cps

        @pl.when(s < n_fill)
        def _init():
            pltpu.sync_copy(bias_hbm.at[c], btile_v.at[pl.ds(0, 1)])
            for j in range(f // 16):
                sl = pl.ds(j * 16, 16)
                bv = btile_v[0, sl]
                for r in range(1, btile):
                    btile_v[r, sl] = bv
            for t in range(rows_per_fill // btile):
                pltpu.sync_copy(
                    btile_v, acc_sh.at[pl.ds(s * rows_per_fill + t * btile, btile)]
                )

        plsc.subcore_barrier()

        def start_edges(k, eb):
            pltpu.async_copy(edata_hbm.at[c, chunk0 + k], ebufs[eb], esem[eb])

        def wait_edges(k, eb):
            pltpu.make_async_copy(
                edata_hbm.at[c, chunk0 + k], ebufs[eb], esem[eb]
            ).wait()

        def start_gather(eb, b):
            pltpu.async_copy(h_hbm.at[ebufs[eb].at[0]], rows[b], gsem[b])

        def wait_gather(eb, b):
            pltpu.make_async_copy(h_hbm.at[ebufs[eb].at[0]], rows[b], gsem[b]).wait()

        def start_scatter(eb, b):
            pltpu.async_copy(rows[b], acc_sh.at[ebufs[eb].at[1]], ssem[b], add=True)

        def wait_scatter(eb, b):
            pltpu.make_async_copy(rows[b], acc_sh.at[ebufs[eb].at[1]], ssem[b]).wait()

        def scale(eb, b):
            ebuf_s = ebufs[eb]
            rows_b = rows[b]

            two = jnp.full((16,), 2, jnp.int32)

            def scale_group(g2, inner):
                base_e = g2 * 16
                for e in range(16):
                    row = base_e + e
                    wi = plsc.load_gather(ebuf_s, [two, jnp.full((16,), row, jnp.int32)])
                    ws = plsc.bitcast(wi, jnp.float32)
                    for j in range(f // 16):
                        sl = pl.ds(j * 16, 16)
                        rows_b[row, sl] = rows_b[row, sl] * ws
                return inner

            lax.fori_loop(0, CHUNK // 16, scale_group, 0)

        start_edges(0, 0)
        start_edges(1, 1)
        wait_edges(0, 0)
        start_gather(0, 0)

        def group_body(g, carry):
            for b in range(GROUP):
                k = g * GROUP + b
                b3 = b % NBUF
                b4 = b % EBUF

                @pl.when(k >= 2)
                def _wait_sc():
                    wait_scatter((b + 2) % EBUF, (b + 1) % NBUF)

                @pl.when(k + 2 < cps)
                def _pref():
                    start_edges(k + 2, (b + 2) % EBUF)

                @pl.when(k + 1 < cps)
                def _next_gather():
                    wait_edges(k + 1, (b + 1) % EBUF)
                    start_gather((b + 1) % EBUF, (b + 1) % NBUF)

                wait_gather(b4, b3)
                scale(b4, b3)
                start_scatter(b4, b3)
            return carry

        lax.fori_loop(0, cps // GROUP, group_body, 0)

        wait_scatter((cps - 2) % EBUF, (cps - 2) % NBUF)
        wait_scatter((cps - 1) % EBUF, (cps - 1) % NBUF)

        plsc.subcore_barrier()

        @pl.when(s < n_fill)
        def _drain():
            r0 = s * rows_per_fill
            pltpu.sync_copy(
                acc_sh.at[pl.ds(r0, rows_per_fill)],
                out_hbm.at[c].at[pl.ds(r0, rows_per_fill)],
            )

    return sc_agg


def _pack_edata(src0, src1, dst, w, n_chunks):
    w_bits = lax.bitcast_convert_type(w, jnp.int32)

    def per_core(s):
        return jnp.stack(
            [
                s.reshape(n_chunks, CHUNK),
                dst.reshape(n_chunks, CHUNK),
                w_bits.reshape(n_chunks, CHUNK),
            ],
            axis=1,
        )

    return jnp.stack([per_core(src0), per_core(src1)])


def kernel(x, edge_index, edge_weight, W1, b1, W2, b2, a, b, c, d):
    n = x.shape[0]
    e = edge_index.shape[1]
    quant = 2 * N_SUBCORES * CHUNK * GROUP
    e_pad = ((e + quant - 1) // quant) * quant
    pad = e_pad - e
    n_chunks = e_pad // CHUNK

    spread = jnp.arange(pad, dtype=jnp.int32) % jnp.int32(n)
    src = jnp.concatenate([edge_index[0], spread])
    dst = jnp.concatenate([edge_index[1], spread])
    w = jnp.concatenate([edge_weight, jnp.zeros((pad,), jnp.float32)])

    edata1 = _pack_edata(src, src + n, dst, w, n_chunks)
    edata2 = _pack_edata(src, src, dst, w, n_chunks)

    h1 = _matmul_split(x, W1)
    agg1 = _make_sc_agg(HID // 2, 2 * n, e_pad, edge_split=False)(
        h1.reshape(2 * n, HID // 2), edata1, b1.reshape(2, 1, HID // 2)
    )

    params = jnp.stack([a, b, c, d])
    h2 = _matmul2_full(params, agg1, W2)
    bias2 = jnp.stack([b2, jnp.zeros_like(b2)]).reshape(2, 1, D_OUT)
    parts = _make_sc_agg(D_OUT, n, e_pad, edge_split=True)(
        h2, edata2, bias2
    )
    return _sum_partials(parts)


# speedup vs baseline: 3.2797x; 1.0287x over previous
import functools

import jax
import jax.numpy as jnp
from jax import lax
from jax.experimental import pallas as pl
from jax.experimental.pallas import tpu as pltpu
from jax.experimental.pallas import tpu_sc as plsc

N_NODES = 10000
D_IN = 128
HID = 256
D_OUT = 128
N_SUBCORES = 16
CHUNK = 112
NBUF = 3
EBUF = 4
GROUP = 12


_BLK_M = 2000


def _mm1_body(x_ref, w_ref, o_ref):
    o_ref[0] = jnp.dot(x_ref[...], w_ref[...], preferred_element_type=jnp.float32)


def _matmul_split(x, w):
    n, k = x.shape
    fh = w.shape[1] // 2
    grid = (n // _BLK_M, 2)
    return pl.pallas_call(
        _mm1_body,
        grid=grid,
        in_specs=[
            pl.BlockSpec((_BLK_M, k), lambda i, c: (i, 0)),
            pl.BlockSpec((k, fh), lambda i, c: (0, c)),
        ],
        out_specs=pl.BlockSpec((1, _BLK_M, fh), lambda i, c: (c, i, 0)),
        out_shape=jax.ShapeDtypeStruct((2, n, fh), jnp.float32),
    )(x, w)


def _grelu(x, ga, gb, gc, gd):
    out = jnp.where(x < 0, ga * x, x)
    out = jnp.where((x >= 0) & (x < gc), gb * x, out)
    out = jnp.where(x >= gc, gd * x, out)
    return out


def _mm2_body(p_ref, agg_ref, w_ref, o_ref):
    ga, gb, gc, gd = p_ref[0], p_ref[1], p_ref[2], p_ref[3]
    a0 = _grelu(agg_ref[0], ga, gb, gc, gd)
    a1 = _grelu(agg_ref[1], ga, gb, gc, gd)
    k = a0.shape[1]
    o_ref[...] = jnp.dot(a0, w_ref[:k, :], preferred_element_type=jnp.float32) + jnp.dot(
        a1, w_ref[k:, :], preferred_element_type=jnp.float32
    )


def _matmul2_full(params, agg, w):
    _, n, kh = agg.shape
    f = w.shape[1]
    grid = (n // _BLK_M,)
    return pl.pallas_call(
        _mm2_body,
        grid=grid,
        in_specs=[
            pl.BlockSpec(memory_space=pltpu.SMEM),
            pl.BlockSpec((2, _BLK_M, kh), lambda i: (0, i, 0)),
            pl.BlockSpec((2 * kh, f), lambda i: (0, 0)),
        ],
        out_specs=pl.BlockSpec((_BLK_M, f), lambda i: (i, 0)),
        out_shape=jax.ShapeDtypeStruct((n, f), jnp.float32),
    )(params, agg, w)


def _sum2_body(in_ref, o_ref):
    o_ref[...] = in_ref[0] + in_ref[1]


def _sum_partials(p):
    _, n, f = p.shape
    grid = (n // _BLK_M,)
    return pl.pallas_call(
        _sum2_body,
        grid=grid,
        in_specs=[pl.BlockSpec((2, _BLK_M, f), lambda i: (0, i, 0))],
        out_specs=pl.BlockSpec((_BLK_M, f), lambda i: (i, 0)),
        out_shape=jax.ShapeDtypeStruct((n, f), jnp.float32),
    )(p)


def _make_sc_agg(f, n_h, e_pad, edge_split):
    n = N_NODES
    n_chunks = e_pad // CHUNK
    cps = n_chunks // (2 * N_SUBCORES if edge_split else N_SUBCORES)
    n_groups = -(-cps // GROUP)
    assert cps >= GROUP
    n_fill = 10
    rows_per_fill = n // n_fill
    btile = 20

    mesh = plsc.VectorSubcoreMesh(core_axis_name="c", subcore_axis_name="s")

    @functools.partial(
        pl.kernel,
        out_type=jax.ShapeDtypeStruct((2, n, f), jnp.float32),
        mesh=mesh,
        compiler_params=pltpu.CompilerParams(needs_layout_passes=False),
        scratch_types=[
            [pltpu.VMEM((3, CHUNK), jnp.int32) for _ in range(EBUF)],
            [pltpu.VMEM((CHUNK, f), jnp.float32) for _ in range(NBUF)],
            pltpu.VMEM_SHARED((n, f), jnp.float32),
            [pltpu.SemaphoreType.DMA for _ in range(EBUF)],
            [pltpu.SemaphoreType.DMA for _ in range(NBUF)],
            [pltpu.SemaphoreType.DMA for _ in range(NBUF)],
        ],
    )
    def sc_agg(h_hbm, edata_hbm, bias_hbm, out_hbm,
               ebufs, rows, acc_sh, esem, gsem, ssem):
        c = lax.axis_index("c")
        s = lax.axis_index("s")

        if edge_split:
            chunk0 = (c * N_SUBCORES + s) * cps
        else:
            chunk0 = s * cps

        @pl.when(s < n_fill)
        def _init():
            btile_v = rows[0]
            pltpu.sync_copy(bias_hbm.at[c], btile_v.at[pl.ds(0, 1)])
            for j in range(f // 16):
                sl = pl.ds(j * 16, 16)
                bv = btile_v[0, sl]
                for r in range(1, btile):
                    btile_v[r, sl] = bv
            for t in range(rows_per_fill // btile):
                pltpu.sync_copy(
                    btile_v.at[pl.ds(0, btile)],
                    acc_sh.at[pl.ds(s * rows_per_fill + t * btile, btile)],
                )

        plsc.subcore_barrier()

        def start_edges(k, eb):
            pltpu.async_copy(edata_hbm.at[c, chunk0 + k], ebufs[eb], esem[eb])

        def wait_edges(k, eb):
            pltpu.make_async_copy(
                edata_hbm.at[c, chunk0 + k], ebufs[eb], esem[eb]
            ).wait()

        def start_gather(eb, b):
            pltpu.async_copy(h_hbm.at[ebufs[eb].at[0]], rows[b], gsem[b])

        def wait_gather(eb, b):
            pltpu.make_async_copy(h_hbm.at[ebufs[eb].at[0]], rows[b], gsem[b]).wait()

        def start_scatter(eb, b):
            pltpu.async_copy(rows[b], acc_sh.at[ebufs[eb].at[1]], ssem[b], add=True)

        def wait_scatter(eb, b):
            pltpu.make_async_copy(rows[b], acc_sh.at[ebufs[eb].at[1]], ssem[b]).wait()

        def scale(eb, b):
            ebuf_s = ebufs[eb]
            rows_b = rows[b]

            two = jnp.full((16,), 2, jnp.int32)

            def scale_group(g2, inner):
                base_e = g2 * 16
                for e in range(16):
                    row = base_e + e
                    wi = plsc.load_gather(ebuf_s, [two, jnp.full((16,), row, jnp.int32)])
                    ws = plsc.bitcast(wi, jnp.float32)
                    for j in range(f // 16):
                        sl = pl.ds(j * 16, 16)
                        rows_b[row, sl] = rows_b[row, sl] * ws
                return inner

            lax.fori_loop(0, CHUNK // 16, scale_group, 0)

        start_edges(0, 0)
        start_edges(1, 1)
        wait_edges(0, 0)
        start_gather(0, 0)

        def group_body(g, carry):
            for b in range(GROUP):
                k = g * GROUP + b
                b3 = b % NBUF
                b4 = b % EBUF

                @pl.when((k >= 2) & (k < cps + 2))
                def _wait_sc():
                    wait_scatter((b + 2) % EBUF, (b + 1) % NBUF)

                @pl.when(k + 2 < cps)
                def _pref():
                    start_edges(k + 2, (b + 2) % EBUF)

                @pl.when(k + 1 < cps)
                def _next_gather():
                    wait_edges(k + 1, (b + 1) % EBUF)
                    start_gather((b + 1) % EBUF, (b + 1) % NBUF)

                @pl.when(k < cps)
                def _work():
                    wait_gather(b4, b3)
                    scale(b4, b3)
                    start_scatter(b4, b3)

            return carry

        lax.fori_loop(0, n_groups, group_body, 0)

        for k_wait in range(max(0, n_groups * GROUP - 2), cps):
            wait_scatter(k_wait % EBUF, k_wait % NBUF)

        plsc.subcore_barrier()

        @pl.when(s < n_fill)
        def _drain():
            r0 = s * rows_per_fill
            pltpu.sync_copy(
                acc_sh.at[pl.ds(r0, rows_per_fill)],
                out_hbm.at[c].at[pl.ds(r0, rows_per_fill)],
            )

    return sc_agg


def _pack_edata(src0, src1, dst, w, n_chunks):
    w_bits = lax.bitcast_convert_type(w, jnp.int32)

    def per_core(s):
        return jnp.stack(
            [
                s.reshape(n_chunks, CHUNK),
                dst.reshape(n_chunks, CHUNK),
                w_bits.reshape(n_chunks, CHUNK),
            ],
            axis=1,
        )

    return jnp.stack([per_core(src0), per_core(src1)])


def kernel(x, edge_index, edge_weight, W1, b1, W2, b2, a, b, c, d):
    n = x.shape[0]
    e = edge_index.shape[1]
    quant = 2 * N_SUBCORES * CHUNK
    e_pad = ((e + quant - 1) // quant) * quant
    pad = e_pad - e
    n_chunks = e_pad // CHUNK

    spread = jnp.arange(pad, dtype=jnp.int32) % jnp.int32(n)
    src = jnp.concatenate([edge_index[0], spread])
    dst = jnp.concatenate([edge_index[1], spread])
    w = jnp.concatenate([edge_weight, jnp.zeros((pad,), jnp.float32)])

    edata = _pack_edata(src, src + n, dst, w, n_chunks)

    h1 = _matmul_split(x, W1)
    agg1 = _make_sc_agg(HID // 2, 2 * n, e_pad, edge_split=False)(
        h1.reshape(2 * n, HID // 2), edata, b1.reshape(2, 1, HID // 2)
    )

    params = jnp.stack([a, b, c, d])
    h2 = _matmul2_full(params, agg1, W2)
    h2cat = jnp.concatenate([h2, h2], axis=0)
    bias2 = jnp.stack([b2, jnp.zeros_like(b2)]).reshape(2, 1, D_OUT)
    parts = _make_sc_agg(D_OUT, 2 * n, e_pad, edge_split=True)(
        h2cat, edata, bias2
    )
    return _sum_partials(parts)


# speedup vs baseline: 3.3173x; 1.0115x over previous
import functools

import jax
import jax.numpy as jnp
from jax import lax
from jax.experimental import pallas as pl
from jax.experimental.pallas import tpu as pltpu
from jax.experimental.pallas import tpu_sc as plsc

N_NODES = 10000
D_IN = 128
HID = 256
D_OUT = 128
N_SUBCORES = 16
CHUNK = 112
NBUF = 3
EBUF = 4
GROUP = 12


_BLK_M = 2000


def _mm1_body(x_ref, w_ref, o_ref):
    o_ref[0] = jnp.dot(x_ref[...], w_ref[...], preferred_element_type=jnp.float32)


def _matmul_split(x, w):
    n, k = x.shape
    fh = w.shape[1] // 2
    grid = (n // _BLK_M, 2)
    return pl.pallas_call(
        _mm1_body,
        grid=grid,
        in_specs=[
            pl.BlockSpec((_BLK_M, k), lambda i, c: (i, 0)),
            pl.BlockSpec((k, fh), lambda i, c: (0, c)),
        ],
        out_specs=pl.BlockSpec((1, _BLK_M, fh), lambda i, c: (c, i, 0)),
        out_shape=jax.ShapeDtypeStruct((2, n, fh), jnp.float32),
    )(x, w)


def _grelu(x, ga, gb, gc, gd):
    out = jnp.where(x < 0, ga * x, x)
    out = jnp.where((x >= 0) & (x < gc), gb * x, out)
    out = jnp.where(x >= gc, gd * x, out)
    return out


def _mm2_body(p_ref, agg_ref, w_ref, o_ref):
    ga, gb, gc, gd = p_ref[0], p_ref[1], p_ref[2], p_ref[3]
    a0 = _grelu(agg_ref[0], ga, gb, gc, gd)
    a1 = _grelu(agg_ref[1], ga, gb, gc, gd)
    k = a0.shape[1]
    o_ref[...] = jnp.dot(a0, w_ref[:k, :], preferred_element_type=jnp.float32) + jnp.dot(
        a1, w_ref[k:, :], preferred_element_type=jnp.float32
    )


def _matmul2_full(params, agg, w):
    _, n, kh = agg.shape
    f = w.shape[1]
    nb = n // _BLK_M
    grid = (nb, 2)
    return pl.pallas_call(
        _mm2_body,
        grid=grid,
        in_specs=[
            pl.BlockSpec(memory_space=pltpu.SMEM),
            pl.BlockSpec((2, _BLK_M, kh), lambda i, r: (0, i, 0)),
            pl.BlockSpec((2 * kh, f), lambda i, r: (0, 0)),
        ],
        out_specs=pl.BlockSpec((_BLK_M, f), lambda i, r: (r * nb + i, 0)),
        out_shape=jax.ShapeDtypeStruct((2 * n, f), jnp.float32),
    )(params, agg, w)


def _sum2_body(in_ref, o_ref):
    o_ref[...] = in_ref[0] + in_ref[1]


def _sum_partials(p):
    _, n, f = p.shape
    grid = (n // _BLK_M,)
    return pl.pallas_call(
        _sum2_body,
        grid=grid,
        in_specs=[pl.BlockSpec((2, _BLK_M, f), lambda i: (0, i, 0))],
        out_specs=pl.BlockSpec((_BLK_M, f), lambda i: (i, 0)),
        out_shape=jax.ShapeDtypeStruct((n, f), jnp.float32),
    )(p)


def _make_sc_agg(f, n_h, e_pad, edge_split):
    n = N_NODES
    n_chunks = e_pad // CHUNK
    cps = n_chunks // (2 * N_SUBCORES if edge_split else N_SUBCORES)
    n_groups = -(-cps // GROUP)
    assert cps >= GROUP
    n_fill = 10
    rows_per_fill = n // n_fill
    btile = 20

    mesh = plsc.VectorSubcoreMesh(core_axis_name="c", subcore_axis_name="s")

    @functools.partial(
        pl.kernel,
        out_type=jax.ShapeDtypeStruct((2, n, f), jnp.float32),
        mesh=mesh,
        compiler_params=pltpu.CompilerParams(needs_layout_passes=False),
        scratch_types=[
            pltpu.VMEM((EBUF, 3, CHUNK), jnp.int32),
            pltpu.VMEM((NBUF, CHUNK, f), jnp.float32),
            pltpu.VMEM_SHARED((n, f), jnp.float32),
            [pltpu.SemaphoreType.DMA for _ in range(EBUF)],
            [pltpu.SemaphoreType.DMA for _ in range(NBUF)],
            [pltpu.SemaphoreType.DMA for _ in range(NBUF)],
        ],
    )
    def sc_agg(h_hbm, edata_hbm, bias_hbm, out_hbm,
               ebuf_all, rows_all, acc_sh, esem, gsem, ssem):
        ebufs = [ebuf_all.at[i] for i in range(EBUF)]
        rows = [rows_all.at[i] for i in range(NBUF)]
        c = lax.axis_index("c")
        s = lax.axis_index("s")

        if edge_split:
            chunk0 = (c * N_SUBCORES + s) * cps
        else:
            chunk0 = s * cps

        @pl.when(s < n_fill)
        def _init():
            btile_v = rows[0]
            pltpu.sync_copy(bias_hbm.at[c], btile_v.at[pl.ds(0, 1)])
            for j in range(f // 16):
                sl = pl.ds(j * 16, 16)
                bv = btile_v[0, sl]
                for r in range(1, btile):
                    btile_v[r, sl] = bv
            for t in range(rows_per_fill // btile):
                pltpu.sync_copy(
                    btile_v.at[pl.ds(0, btile)],
                    acc_sh.at[pl.ds(s * rows_per_fill + t * btile, btile)],
                )

        plsc.subcore_barrier()

        def start_edges(k, eb):
            pltpu.async_copy(edata_hbm.at[c, chunk0 + k], ebufs[eb], esem[eb])

        def wait_edges(k, eb):
            pltpu.make_async_copy(
                edata_hbm.at[c, chunk0 + k], ebufs[eb], esem[eb]
            ).wait()

        def start_gather(eb, b):
            pltpu.async_copy(h_hbm.at[ebufs[eb].at[0]], rows[b], gsem[b])

        def wait_gather(eb, b):
            pltpu.make_async_copy(h_hbm.at[ebufs[eb].at[0]], rows[b], gsem[b]).wait()

        def start_scatter(eb, b):
            pltpu.async_copy(rows[b], acc_sh.at[ebufs[eb].at[1]], ssem[b], add=True)

        def wait_scatter(eb, b):
            pltpu.make_async_copy(rows[b], acc_sh.at[ebufs[eb].at[1]], ssem[b]).wait()

        def scale(eb, b):
            ebuf_s = ebufs[eb]
            rows_b = rows[b]

            two = jnp.full((16,), 2, jnp.int32)

            def scale_group(g2, inner):
                base_e = g2 * 16
                for e in range(16):
                    row = base_e + e
                    wi = plsc.load_gather(ebuf_s, [two, jnp.full((16,), row, jnp.int32)])
                    ws = plsc.bitcast(wi, jnp.float32)
                    for j in range(f // 16):
                        sl = pl.ds(j * 16, 16)
                        rows_b[row, sl] = rows_b[row, sl] * ws
                return inner

            lax.fori_loop(0, CHUNK // 16, scale_group, 0)

        start_edges(0, 0)
        start_edges(1, 1)
        wait_edges(0, 0)
        start_gather(0, 0)

        def group_body(g, carry):
            for b in range(GROUP):
                k = g * GROUP + b
                b3 = b % NBUF
                b4 = b % EBUF

                @pl.when((k >= 2) & (k < cps + 2))
                def _wait_sc():
                    wait_scatter((b + 2) % EBUF, (b + 1) % NBUF)

                @pl.when(k + 2 < cps)
                def _pref():
                    start_edges(k + 2, (b + 2) % EBUF)

                @pl.when(k + 1 < cps)
                def _next_gather():
                    wait_edges(k + 1, (b + 1) % EBUF)
                    start_gather((b + 1) % EBUF, (b + 1) % NBUF)

                @pl.when(k < cps)
                def _work():
                    wait_gather(b4, b3)
                    scale(b4, b3)
                    start_scatter(b4, b3)

            return carry

        lax.fori_loop(0, n_groups, group_body, 0)

        for k_wait in range(max(0, n_groups * GROUP - 2), cps):
            wait_scatter(k_wait % EBUF, k_wait % NBUF)

        plsc.subcore_barrier()

        @pl.when(s < n_fill)
        def _drain():
            r0 = s * rows_per_fill
            pltpu.sync_copy(
                acc_sh.at[pl.ds(r0, rows_per_fill)],
                out_hbm.at[c].at[pl.ds(r0, rows_per_fill)],
            )

    return sc_agg


def _pack_edata(src0, src1, dst, w, n_chunks):
    w_bits = lax.bitcast_convert_type(w, jnp.int32)

    def per_core(s):
        return jnp.stack(
            [
                s.reshape(n_chunks, CHUNK),
                dst.reshape(n_chunks, CHUNK),
                w_bits.reshape(n_chunks, CHUNK),
            ],
            axis=1,
        )

    return jnp.stack([per_core(src0), per_core(src1)])


def kernel(x, edge_index, edge_weight, W1, b1, W2, b2, a, b, c, d):
    n = x.shape[0]
    e = edge_index.shape[1]
    quant = 2 * N_SUBCORES * CHUNK
    e_pad = ((e + quant - 1) // quant) * quant
    pad = e_pad - e
    n_chunks = e_pad // CHUNK

    spread = jnp.arange(pad, dtype=jnp.int32) % jnp.int32(n)
    src = jnp.concatenate([edge_index[0], spread])
    dst = jnp.concatenate([edge_index[1], spread])
    w = jnp.concatenate([edge_weight, jnp.zeros((pad,), jnp.float32)])

    edata = _pack_edata(src, src + n, dst, w, n_chunks)

    h1 = _matmul_split(x, W1)
    agg1 = _make_sc_agg(HID // 2, 2 * n, e_pad, edge_split=False)(
        h1.reshape(2 * n, HID // 2), edata, b1.reshape(2, 1, HID // 2)
    )

    params = jnp.stack([a, b, c, d])
    h2cat = _matmul2_full(params, agg1, W2)
    bias2 = jnp.stack([b2, jnp.zeros_like(b2)]).reshape(2, 1, D_OUT)
    parts = _make_sc_agg(D_OUT, 2 * n, e_pad, edge_split=True)(
        h2cat, edata, bias2
    )
    return _sum_partials(parts)


# speedup vs baseline: 3.3549x; 1.0113x over previous
import functools

import jax
import jax.numpy as jnp
from jax import lax
from jax.experimental import pallas as pl
from jax.experimental.pallas import tpu as pltpu
from jax.experimental.pallas import tpu_sc as plsc

N_NODES = 10000
D_IN = 128
HID = 256
D_OUT = 128
N_SUBCORES = 16
CHUNK = 112
NBUF = 3
EBUF = 4
GROUP = 12


_BLK_M = 2000


def _mm1_body(x_ref, w_ref, o_ref):
    o_ref[0] = jnp.dot(x_ref[...], w_ref[...], preferred_element_type=jnp.float32)


def _matmul_split(x, w):
    n, k = x.shape
    fh = w.shape[1] // 2
    grid = (n // _BLK_M, 2)
    return pl.pallas_call(
        _mm1_body,
        grid=grid,
        in_specs=[
            pl.BlockSpec((_BLK_M, k), lambda i, c: (i, 0)),
            pl.BlockSpec((k, fh), lambda i, c: (0, c)),
        ],
        out_specs=pl.BlockSpec((1, _BLK_M, fh), lambda i, c: (c, i, 0)),
        out_shape=jax.ShapeDtypeStruct((2, n, fh), jnp.float32),
    )(x, w)


def _grelu(x, ga, gb, gc, gd):
    out = jnp.where(x < 0, ga * x, x)
    out = jnp.where((x >= 0) & (x < gc), gb * x, out)
    out = jnp.where(x >= gc, gd * x, out)
    return out


def _mm2_body(p_ref, agg_ref, w_ref, o_ref):
    ga, gb, gc, gd = p_ref[0], p_ref[1], p_ref[2], p_ref[3]
    a0 = _grelu(agg_ref[0], ga, gb, gc, gd)
    a1 = _grelu(agg_ref[1], ga, gb, gc, gd)
    k = a0.shape[1]
    o_ref[...] = jnp.dot(a0, w_ref[:k, :], preferred_element_type=jnp.float32) + jnp.dot(
        a1, w_ref[k:, :], preferred_element_type=jnp.float32
    )


def _matmul2_full(params, agg, w):
    _, n, kh = agg.shape
    f = w.shape[1]
    nb = n // _BLK_M
    grid = (nb, 2)
    return pl.pallas_call(
        _mm2_body,
        grid=grid,
        in_specs=[
            pl.BlockSpec(memory_space=pltpu.SMEM),
            pl.BlockSpec((2, _BLK_M, kh), lambda i, r: (0, i, 0)),
            pl.BlockSpec((2 * kh, f), lambda i, r: (0, 0)),
        ],
        out_specs=pl.BlockSpec((_BLK_M, f), lambda i, r: (r * nb + i, 0)),
        out_shape=jax.ShapeDtypeStruct((2 * n, f), jnp.float32),
    )(params, agg, w)


def _sum2_body(in_ref, o_ref):
    o_ref[...] = in_ref[0] + in_ref[1]


def _sum_partials(p):
    _, n, f = p.shape
    grid = (n // _BLK_M,)
    return pl.pallas_call(
        _sum2_body,
        grid=grid,
        in_specs=[pl.BlockSpec((2, _BLK_M, f), lambda i: (0, i, 0))],
        out_specs=pl.BlockSpec((_BLK_M, f), lambda i: (i, 0)),
        out_shape=jax.ShapeDtypeStruct((n, f), jnp.float32),
    )(p)


def _make_sc_agg(f, n_h, e_pad, edge_split):
    n = N_NODES
    n_chunks = e_pad // CHUNK
    cps = n_chunks // (2 * N_SUBCORES if edge_split else N_SUBCORES)
    n_groups = -(-cps // GROUP)
    assert cps >= GROUP
    n_fill = 10
    rows_per_fill = n // n_fill
    btile = 20

    mesh = plsc.VectorSubcoreMesh(core_axis_name="c", subcore_axis_name="s")

    @functools.partial(
        pl.kernel,
        out_type=jax.ShapeDtypeStruct((2, n, f), jnp.float32),
        mesh=mesh,
        compiler_params=pltpu.CompilerParams(needs_layout_passes=False),
        scratch_types=[
            pltpu.VMEM((EBUF, 3, CHUNK), jnp.int32),
            pltpu.VMEM((NBUF, CHUNK, f), jnp.float32),
            pltpu.VMEM_SHARED((n, f), jnp.float32),
            [pltpu.SemaphoreType.DMA for _ in range(EBUF)],
            [pltpu.SemaphoreType.DMA for _ in range(NBUF)],
            [pltpu.SemaphoreType.DMA for _ in range(NBUF)],
            pltpu.SemaphoreType.DMA,
        ],
    )
    def sc_agg(h_hbm, edata_hbm, bias_hbm, out_hbm,
               ebuf_all, rows_all, acc_sh, esem, gsem, ssem, isem):
        ebufs = [ebuf_all.at[i] for i in range(EBUF)]
        rows = [rows_all.at[i] for i in range(NBUF)]
        c = lax.axis_index("c")
        s = lax.axis_index("s")

        if edge_split:
            chunk0 = (c * N_SUBCORES + s) * cps
        else:
            chunk0 = s * cps

        @pl.when(s < n_fill)
        def _init():
            btile_v = rows[0]
            pltpu.sync_copy(bias_hbm.at[c], btile_v.at[pl.ds(0, 1)])
            for j in range(f // 16):
                sl = pl.ds(j * 16, 16)
                bv = btile_v[0, sl]
                for r in range(1, btile):
                    btile_v[r, sl] = bv
            for t in range(rows_per_fill // btile):
                pltpu.async_copy(
                    btile_v.at[pl.ds(0, btile)],
                    acc_sh.at[pl.ds(s * rows_per_fill + t * btile, btile)],
                    isem,
                )
            for t in range(rows_per_fill // btile):
                pltpu.make_async_copy(
                    btile_v.at[pl.ds(0, btile)],
                    acc_sh.at[pl.ds(s * rows_per_fill + t * btile, btile)],
                    isem,
                ).wait()

        plsc.subcore_barrier()

        def start_edges(k, eb):
            pltpu.async_copy(edata_hbm.at[c, chunk0 + k], ebufs[eb], esem[eb])

        def wait_edges(k, eb):
            pltpu.make_async_copy(
                edata_hbm.at[c, chunk0 + k], ebufs[eb], esem[eb]
            ).wait()

        def start_gather(eb, b):
            pltpu.async_copy(h_hbm.at[ebufs[eb].at[0]], rows[b], gsem[b])

        def wait_gather(eb, b):
            pltpu.make_async_copy(h_hbm.at[ebufs[eb].at[0]], rows[b], gsem[b]).wait()

        def start_scatter(eb, b):
            pltpu.async_copy(rows[b], acc_sh.at[ebufs[eb].at[1]], ssem[b], add=True)

        def wait_scatter(eb, b):
            pltpu.make_async_copy(rows[b], acc_sh.at[ebufs[eb].at[1]], ssem[b]).wait()

        def scale(eb, b):
            ebuf_s = ebufs[eb]
            rows_b = rows[b]

            two = jnp.full((16,), 2, jnp.int32)

            def scale_group(g2, inner):
                base_e = g2 * 16
                for e in range(16):
                    row = base_e + e
                    wi = plsc.load_gather(ebuf_s, [two, jnp.full((16,), row, jnp.int32)])
                    ws = plsc.bitcast(wi, jnp.float32)
                    for j in range(f // 16):
                        sl = pl.ds(j * 16, 16)
                        rows_b[row, sl] = rows_b[row, sl] * ws
                return inner

            lax.fori_loop(0, CHUNK // 16, scale_group, 0)

        start_edges(0, 0)
        start_edges(1, 1)
        wait_edges(0, 0)
        start_gather(0, 0)

        def group_body(g, carry):
            for b in range(GROUP):
                k = g * GROUP + b
                b3 = b % NBUF
                b4 = b % EBUF

                @pl.when((k >= 2) & (k < cps + 2))
                def _wait_sc():
                    wait_scatter((b + 2) % EBUF, (b + 1) % NBUF)

                @pl.when(k + 2 < cps)
                def _pref():
                    start_edges(k + 2, (b + 2) % EBUF)

                @pl.when(k + 1 < cps)
                def _next_gather():
                    wait_edges(k + 1, (b + 1) % EBUF)
                    start_gather((b + 1) % EBUF, (b + 1) % NBUF)

                @pl.when(k < cps)
                def _work():
                    wait_gather(b4, b3)
                    scale(b4, b3)
                    start_scatter(b4, b3)

            return carry

        lax.fori_loop(0, n_groups, group_body, 0)

        for k_wait in range(max(0, n_groups * GROUP - 2), cps):
            wait_scatter(k_wait % EBUF, k_wait % NBUF)

        plsc.subcore_barrier()

        @pl.when(s < n_fill)
        def _drain():
            r0 = s * rows_per_fill
            pltpu.sync_copy(
                acc_sh.at[pl.ds(r0, rows_per_fill)],
                out_hbm.at[c].at[pl.ds(r0, rows_per_fill)],
            )

    return sc_agg


def _pack_edata(src0, src1, dst, w, n_chunks):
    w_bits = lax.bitcast_convert_type(w, jnp.int32)

    def per_core(s):
        return jnp.stack(
            [
                s.reshape(n_chunks, CHUNK),
                dst.reshape(n_chunks, CHUNK),
                w_bits.reshape(n_chunks, CHUNK),
            ],
            axis=1,
        )

    return jnp.stack([per_core(src0), per_core(src1)])


def kernel(x, edge_index, edge_weight, W1, b1, W2, b2, a, b, c, d):
    n = x.shape[0]
    e = edge_index.shape[1]
    quant = 2 * N_SUBCORES * CHUNK
    e_pad = ((e + quant - 1) // quant) * quant
    pad = e_pad - e
    n_chunks = e_pad // CHUNK

    spread = jnp.arange(pad, dtype=jnp.int32) % jnp.int32(n)
    src = jnp.concatenate([edge_index[0], spread])
    dst = jnp.concatenate([edge_index[1], spread])
    w = jnp.concatenate([edge_weight, jnp.zeros((pad,), jnp.float32)])

    edata = _pack_edata(src, src + n, dst, w, n_chunks)

    h1 = _matmul_split(x, W1)
    agg1 = _make_sc_agg(HID // 2, 2 * n, e_pad, edge_split=False)(
        h1.reshape(2 * n, HID // 2), edata, b1.reshape(2, 1, HID // 2)
    )

    params = jnp.stack([a, b, c, d])
    h2cat = _matmul2_full(params, agg1, W2)
    bias2 = jnp.stack([b2, jnp.zeros_like(b2)]).reshape(2, 1, D_OUT)
    parts = _make_sc_agg(D_OUT, 2 * n, e_pad, edge_split=True)(
        h2cat, edata, bias2
    )
    return _sum_partials(parts)
